# baseline (speedup 1.0000x reference)
import functools

import jax
import jax.numpy as jnp
from jax import lax
from jax.experimental import pallas as pl
from jax.experimental.pallas import tpu as pltpu
from jax.experimental.pallas import tpu_sc as plsc

N = 10000
D = 128
G = 64
E = 320000

NPAD = 10240
NTILES = 32
CHUNK = 64
NCHUNK = 159
NBUF = 3
EPT = NCHUNK * CHUNK
EPAD = NTILES * EPT
RPS = NPAD // 16

_mesh = plsc.VectorSubcoreMesh(
    core_axis_name="c", subcore_axis_name="s", num_cores=2, num_subcores=16)


_NR = NPAD // 128


def _deg_body(combo, out0, out1, acc, cbuf, hist, idrow):
    c = lax.axis_index("c")
    s = lax.axis_index("s")
    wid = s * 2 + c
    pltpu.sync_copy(combo.at[wid], cbuf)

    zero16 = jnp.zeros((16,), jnp.float32)

    def zrow(i, _):
        for k in range(8):
            hist[i, pl.ds(k * 16, 16)] = zero16
        return ()
    lax.fori_loop(0, _NR, zrow, ())
    for k in range(_NR // 16):
        idrow[0, pl.ds(k * 16, 16)] = (
            jnp.arange(16, dtype=jnp.int32) + k * 16)
    @pl.when(s < _NR // 8)
    def _():
        pltpu.sync_copy(hist.at[pl.ds(0, 8)], acc.at[pl.ds(s * 8, 8)])
    plsc.subcore_barrier()

    ones16 = jnp.ones((16,), jnp.float32)

    def brow(j, _):
        for k in range(CHUNK // 16):
            dv = lax.shift_right_logical(cbuf[j, pl.ds(k * 16, 16)], 16)
            row = lax.shift_right_logical(dv, 7)
            col = lax.bitwise_and(dv, 127)
            plsc.addupdate_scatter(hist, [row, col], ones16)
        return ()
    lax.fori_loop(0, NCHUNK, brow, ())

    pltpu.sync_copy(hist, acc.at[idrow.at[0]], add=True)
    plsc.subcore_barrier()

    r0 = s * 8

    @pl.when(jnp.logical_and(c == 0, s < _NR // 8))
    def _():
        pltpu.sync_copy(acc.at[pl.ds(r0, 8)], out0.at[pl.ds(r0, 8)])

    @pl.when(jnp.logical_and(c == 1, s < _NR // 8))
    def _():
        pltpu.sync_copy(acc.at[pl.ds(r0, 8)], out1.at[pl.ds(r0, 8)])


_deg_kernel = functools.partial(
    pl.kernel,
    out_type=(jax.ShapeDtypeStruct((_NR, 128), jnp.float32),
              jax.ShapeDtypeStruct((_NR, 128), jnp.float32)),
    mesh=_mesh,
    scratch_types=[
        pltpu.VMEM_SHARED((_NR, 128), jnp.float32),
        pltpu.VMEM((NCHUNK, CHUNK), jnp.int32),
        pltpu.VMEM((_NR, 128), jnp.float32),
        pltpu.VMEM((1, _NR), jnp.int32),
    ],
    compiler_params=pltpu.CompilerParams(needs_layout_passes=False),
)(_deg_body)


def _spmm_body(table, combo, out0, out1, acc, cbuf, sidx, didx,
               gbufs, gsems, ssems):
    c = lax.axis_index("c")
    s = lax.axis_index("s")
    wid = s * 2 + c
    pltpu.sync_copy(combo.at[wid], cbuf)

    zero16 = jnp.zeros((16,), jnp.float32)

    def zrow(i, _):
        for k in range(8):
            gbufs[0][i, pl.ds(k * 16, 16)] = zero16
        return ()
    lax.fori_loop(0, CHUNK, zrow, ())
    for k in range(RPS // CHUNK):
        pltpu.sync_copy(gbufs[0], acc.at[pl.ds(s * RPS + k * CHUNK, CHUNK)])

    def unpack(j, k):
        for q in range(CHUNK // 16):
            v = cbuf[j, pl.ds(q * 16, 16)]
            sidx[k, pl.ds(q * 16, 16)] = lax.bitwise_and(v, 0xFFFF)
            didx[k, pl.ds(q * 16, 16)] = lax.shift_right_logical(v, 16)

    for k in range(2):
        unpack(k, k)
        pltpu.async_copy(table.at[sidx.at[k]], gbufs[k], gsems[k])
    plsc.subcore_barrier()

    def body(i, _):
        for k in range(NBUF):
            jv = i * NBUF + k
            kp = (k + 2) % NBUF

            @pl.when(jv >= 1)
            def _():
                pltpu.make_async_copy(
                    gbufs[kp], acc.at[didx.at[kp]], ssems[kp]).wait()

            @pl.when(jv + 2 < NCHUNK)
            def _():
                unpack(jv + 2, kp)
                pltpu.async_copy(
                    table.at[sidx.at[kp]], gbufs[kp], gsems[kp])

            pltpu.make_async_copy(
                table.at[sidx.at[k]], gbufs[k], gsems[k]).wait()
            pltpu.async_copy(
                gbufs[k], acc.at[didx.at[k]], ssems[k], add=True)
        return ()
    lax.fori_loop(0, NCHUNK // NBUF, body, ())
    k_last = (NCHUNK - 1) % NBUF
    pltpu.make_async_copy(
        gbufs[k_last], acc.at[didx.at[k_last]], ssems[k_last]).wait()
    plsc.subcore_barrier()

    r0 = s * RPS

    @pl.when(c == 0)
    def _():
        pltpu.sync_copy(acc.at[pl.ds(r0, RPS)], out0.at[pl.ds(r0, RPS)])

    @pl.when(c == 1)
    def _():
        pltpu.sync_copy(acc.at[pl.ds(r0, RPS)], out1.at[pl.ds(r0, RPS)])


_spmm_kernel = functools.partial(
    pl.kernel,
    out_type=(jax.ShapeDtypeStruct((NPAD, D), jnp.float32),
              jax.ShapeDtypeStruct((NPAD, D), jnp.float32)),
    mesh=_mesh,
    scratch_types=[
        pltpu.VMEM_SHARED((NPAD, D), jnp.float32),
        pltpu.VMEM((NCHUNK, CHUNK), jnp.int32),
        pltpu.VMEM((NBUF, CHUNK), jnp.int32),
        pltpu.VMEM((NBUF, CHUNK), jnp.int32),
        [pltpu.VMEM((CHUNK, D), jnp.float32)] * NBUF,
        [pltpu.SemaphoreType.DMA] * NBUF,
        [pltpu.SemaphoreType.DMA] * NBUF,
    ],
    compiler_params=pltpu.CompilerParams(needs_layout_passes=False),
)(_spmm_body)


_BLK = 1280
_GRID = NPAD // _BLK


def _first_body(x_ref, dinv_ref, w_ref, o_ref):
    o_ref[...] = jnp.dot(x_ref[...], w_ref[...],
                         preferred_element_type=jnp.float32) * dinv_ref[...]


_first_kernel = pl.pallas_call(
    _first_body,
    grid=(_GRID,),
    in_specs=[pl.BlockSpec((_BLK, D), lambda i: (i, 0)),
              pl.BlockSpec((_BLK, D), lambda i: (i, 0)),
              pl.BlockSpec((D, D), lambda i: (0, 0))],
    out_specs=pl.BlockSpec((_BLK, D), lambda i: (i, 0)),
    out_shape=jax.ShapeDtypeStruct((NPAD, D), jnp.float32),
)


def _mid_body(p0_ref, p1_ref, t_ref, dinv_ref, b_ref, w_ref, o_ref):
    h = jnp.maximum(
        (p0_ref[...] + p1_ref[...] + t_ref[...]) * dinv_ref[...]
        + b_ref[...], 0.0)
    o_ref[...] = jnp.dot(h, w_ref[...],
                         preferred_element_type=jnp.float32) * dinv_ref[...]


_mid_kernel = pl.pallas_call(
    _mid_body,
    grid=(_GRID,),
    in_specs=[pl.BlockSpec((_BLK, D), lambda i: (i, 0)),
              pl.BlockSpec((_BLK, D), lambda i: (i, 0)),
              pl.BlockSpec((_BLK, D), lambda i: (i, 0)),
              pl.BlockSpec((_BLK, D), lambda i: (i, 0)),
              pl.BlockSpec((1, D), lambda i: (0, 0)),
              pl.BlockSpec((D, D), lambda i: (0, 0))],
    out_specs=pl.BlockSpec((_BLK, D), lambda i: (i, 0)),
    out_shape=jax.ShapeDtypeStruct((NPAD, D), jnp.float32),
)


def _pool_body(p0_ref, p1_ref, t_ref, dinv_ref, b_ref, bt_ref, o_ref,
               sums, cnt):
    i = pl.program_id(0)

    @pl.when(i == 0)
    def _():
        sums[...] = jnp.zeros_like(sums)
        cnt[...] = jnp.zeros_like(cnt)

    h = jnp.maximum(
        (p0_ref[...] + p1_ref[...] + t_ref[...]) * dinv_ref[...]
        + b_ref[...], 0.0)
    gid = lax.broadcasted_iota(jnp.int32, (_BLK, 128), 1)
    oh = (bt_ref[...] == gid).astype(jnp.float32)
    dn = (((0,), (0,)), ((), ()))
    sums[...] += lax.dot_general(oh, h, dn,
                                 preferred_element_type=jnp.float32)
    cnt[...] += lax.dot_general(oh, jnp.ones((_BLK, 128), jnp.float32), dn,
                                preferred_element_type=jnp.float32)

    @pl.when(i == _GRID - 1)
    def _():
        o_ref[...] = sums[...] / jnp.maximum(cnt[...], 1.0)


_pool_kernel = pl.pallas_call(
    _pool_body,
    grid=(_GRID,),
    in_specs=[pl.BlockSpec((_BLK, D), lambda i: (i, 0)),
              pl.BlockSpec((_BLK, D), lambda i: (i, 0)),
              pl.BlockSpec((_BLK, D), lambda i: (i, 0)),
              pl.BlockSpec((_BLK, D), lambda i: (i, 0)),
              pl.BlockSpec((1, D), lambda i: (0, 0)),
              pl.BlockSpec((_BLK, 128), lambda i: (i, 0))],
    out_specs=pl.BlockSpec((128, 128), lambda i: (0, 0)),
    out_shape=jax.ShapeDtypeStruct((128, 128), jnp.float32),
    scratch_shapes=[pltpu.VMEM((128, 128), jnp.float32),
                    pltpu.VMEM((128, 128), jnp.float32)],
)


def kernel(x, W1, b1, W2, b2, W3, b3, W4, b4, edge_index, batch):
    npad_e = EPAD - E
    pad_idx = (N + jnp.arange(npad_e, dtype=jnp.int32) % (NPAD - N))
    src_flat = jnp.concatenate([edge_index[0], pad_idx])
    dst_flat = jnp.concatenate([edge_index[1], pad_idx])
    combo = (src_flat | (dst_flat << 16)).reshape(NTILES, NCHUNK, CHUNK)

    x_pad = jnp.zeros((NPAD, D), jnp.float32).at[:N].set(x)
    batch_pad = jnp.full((NPAD,), G, jnp.int32).at[:N].set(batch)
    batch_b = jnp.broadcast_to(batch_pad[:, None], (NPAD, 128))

    deg0, deg1 = _deg_kernel(combo)
    deg = (deg0 + deg1).reshape(NPAD) + 1.0
    dinv = jnp.where(deg > 0, lax.rsqrt(jnp.maximum(deg, 1e-30)), 0.0)
    dinv = jnp.where(jnp.arange(NPAD) < N, dinv, 0.0)
    dinv_b = jnp.broadcast_to(dinv[:, None], (NPAD, D))

    t = _first_kernel(x_pad, dinv_b, W1)
    for (b, Wn) in ((b1, W2), (b2, W3), (b3, W4)):
        p0, p1 = _spmm_kernel(t, combo)
        t = _mid_kernel(p0, p1, t, dinv_b, b.reshape(1, D), Wn)
    p0, p1 = _spmm_kernel(t, combo)
    hg_full = _pool_kernel(p0, p1, t, dinv_b, b4.reshape(1, D), batch_b)
    return hg_full[:G]


# speedup vs baseline: 1.0231x; 1.0231x over previous
import functools

import jax
import jax.numpy as jnp
from jax import lax
from jax.experimental import pallas as pl
from jax.experimental.pallas import tpu as pltpu
from jax.experimental.pallas import tpu_sc as plsc

N = 10000
D = 128
G = 64
E = 320000

NPAD = 10240
NTILES = 32
CHUNK = 64
NCHUNK = 159
NBUF = 3
EPT = NCHUNK * CHUNK
EPAD = NTILES * EPT
RPS = NPAD // 16

_mesh = plsc.VectorSubcoreMesh(
    core_axis_name="c", subcore_axis_name="s", num_cores=2, num_subcores=16)


def _deg_body(combo, out, cbuf, hist):
    c = lax.axis_index("c")
    s = lax.axis_index("s")
    wid = s * 2 + c
    pltpu.sync_copy(combo.at[wid], cbuf)

    zero16 = jnp.zeros((16,), jnp.float32)

    def zrow(i, _):
        hist[pl.ds(i * 16, 16)] = zero16
        return ()
    lax.fori_loop(0, NPAD // 16, zrow, ())

    ones16 = jnp.ones((16,), jnp.float32)

    def brow(j, _):
        for k in range(CHUNK // 16):
            dv = lax.shift_right_logical(cbuf[j, pl.ds(k * 16, 16)], 16)
            plsc.addupdate_scatter(hist, [dv], ones16)
        return ()
    lax.fori_loop(0, NCHUNK, brow, ())

    pltpu.sync_copy(hist, out.at[wid])


_deg_kernel = functools.partial(
    pl.kernel,
    out_type=jax.ShapeDtypeStruct((NTILES, NPAD), jnp.float32),
    mesh=_mesh,
    scratch_types=[
        pltpu.VMEM((NCHUNK, CHUNK), jnp.int32),
        pltpu.VMEM((NPAD,), jnp.float32),
    ],
    compiler_params=pltpu.CompilerParams(needs_layout_passes=False),
)(_deg_body)


def _spmm_body(table, combo, out0, out1, acc, cbuf, sidx, didx,
               gbufs, gsems, ssems):
    c = lax.axis_index("c")
    s = lax.axis_index("s")
    wid = s * 2 + c
    pltpu.sync_copy(combo.at[wid], cbuf)

    zero16 = jnp.zeros((16,), jnp.float32)

    def zrow(i, _):
        for k in range(8):
            gbufs[0][i, pl.ds(k * 16, 16)] = zero16
        return ()
    lax.fori_loop(0, CHUNK, zrow, ())
    for k in range(RPS // CHUNK):
        pltpu.sync_copy(gbufs[0], acc.at[pl.ds(s * RPS + k * CHUNK, CHUNK)])

    def unpack(j, k):
        for q in range(CHUNK // 16):
            v = cbuf[j, pl.ds(q * 16, 16)]
            sidx[k, pl.ds(q * 16, 16)] = lax.bitwise_and(v, 0xFFFF)
            didx[k, pl.ds(q * 16, 16)] = lax.shift_right_logical(v, 16)

    for k in range(2):
        unpack(k, k)
        pltpu.async_copy(table.at[sidx.at[k]], gbufs[k], gsems[k])
    plsc.subcore_barrier()

    def body(i, _):
        for k in range(NBUF):
            jv = i * NBUF + k
            kp = (k + 2) % NBUF

            pltpu.make_async_copy(
                table.at[sidx.at[k]], gbufs[k], gsems[k]).wait()
            pltpu.async_copy(
                gbufs[k], acc.at[didx.at[k]], ssems[k], add=True)

            @pl.when(jv >= 1)
            def _():
                pltpu.make_async_copy(
                    gbufs[kp], acc.at[didx.at[kp]], ssems[kp]).wait()

            @pl.when(jv + 2 < NCHUNK)
            def _():
                unpack(jv + 2, kp)
                pltpu.async_copy(
                    table.at[sidx.at[kp]], gbufs[kp], gsems[kp])
        return ()
    lax.fori_loop(0, NCHUNK // NBUF, body, ())
    k_last = (NCHUNK - 1) % NBUF
    pltpu.make_async_copy(
        gbufs[k_last], acc.at[didx.at[k_last]], ssems[k_last]).wait()
    plsc.subcore_barrier()

    r0 = s * RPS

    @pl.when(c == 0)
    def _():
        pltpu.sync_copy(acc.at[pl.ds(r0, RPS)], out0.at[pl.ds(r0, RPS)])

    @pl.when(c == 1)
    def _():
        pltpu.sync_copy(acc.at[pl.ds(r0, RPS)], out1.at[pl.ds(r0, RPS)])


_spmm_kernel = functools.partial(
    pl.kernel,
    out_type=(jax.ShapeDtypeStruct((NPAD, D), jnp.float32),
              jax.ShapeDtypeStruct((NPAD, D), jnp.float32)),
    mesh=_mesh,
    scratch_types=[
        pltpu.VMEM_SHARED((NPAD, D), jnp.float32),
        pltpu.VMEM((NCHUNK, CHUNK), jnp.int32),
        pltpu.VMEM((NBUF, CHUNK), jnp.int32),
        pltpu.VMEM((NBUF, CHUNK), jnp.int32),
        [pltpu.VMEM((CHUNK, D), jnp.float32)] * NBUF,
        [pltpu.SemaphoreType.DMA] * NBUF,
        [pltpu.SemaphoreType.DMA] * NBUF,
    ],
    compiler_params=pltpu.CompilerParams(needs_layout_passes=False),
)(_spmm_body)


_BLK = 1280
_GRID = NPAD // _BLK


def _first_body(x_ref, dinv_ref, w_ref, o_ref):
    o_ref[...] = jnp.dot(x_ref[...], w_ref[...],
                         preferred_element_type=jnp.float32) * dinv_ref[...]


_first_kernel = pl.pallas_call(
    _first_body,
    grid=(_GRID,),
    in_specs=[pl.BlockSpec((_BLK, D), lambda i: (i, 0)),
              pl.BlockSpec((_BLK, D), lambda i: (i, 0)),
              pl.BlockSpec((D, D), lambda i: (0, 0))],
    out_specs=pl.BlockSpec((_BLK, D), lambda i: (i, 0)),
    out_shape=jax.ShapeDtypeStruct((NPAD, D), jnp.float32),
)


def _mid_body(p0_ref, p1_ref, t_ref, dinv_ref, b_ref, w_ref, o_ref):
    h = jnp.maximum(
        (p0_ref[...] + p1_ref[...] + t_ref[...]) * dinv_ref[...]
        + b_ref[...], 0.0)
    o_ref[...] = jnp.dot(h, w_ref[...],
                         preferred_element_type=jnp.float32) * dinv_ref[...]


_mid_kernel = pl.pallas_call(
    _mid_body,
    grid=(_GRID,),
    in_specs=[pl.BlockSpec((_BLK, D), lambda i: (i, 0)),
              pl.BlockSpec((_BLK, D), lambda i: (i, 0)),
              pl.BlockSpec((_BLK, D), lambda i: (i, 0)),
              pl.BlockSpec((_BLK, D), lambda i: (i, 0)),
              pl.BlockSpec((1, D), lambda i: (0, 0)),
              pl.BlockSpec((D, D), lambda i: (0, 0))],
    out_specs=pl.BlockSpec((_BLK, D), lambda i: (i, 0)),
    out_shape=jax.ShapeDtypeStruct((NPAD, D), jnp.float32),
)


def _pool_body(p0_ref, p1_ref, t_ref, dinv_ref, b_ref, bt_ref, o_ref,
               sums, cnt):
    i = pl.program_id(0)

    @pl.when(i == 0)
    def _():
        sums[...] = jnp.zeros_like(sums)
        cnt[...] = jnp.zeros_like(cnt)

    h = jnp.maximum(
        (p0_ref[...] + p1_ref[...] + t_ref[...]) * dinv_ref[...]
        + b_ref[...], 0.0)
    gid = lax.broadcasted_iota(jnp.int32, (_BLK, 128), 1)
    oh = (bt_ref[...] == gid).astype(jnp.float32)
    dn = (((0,), (0,)), ((), ()))
    sums[...] += lax.dot_general(oh, h, dn,
                                 preferred_element_type=jnp.float32)
    cnt[...] += lax.dot_general(oh, jnp.ones((_BLK, 128), jnp.float32), dn,
                                preferred_element_type=jnp.float32)

    @pl.when(i == _GRID - 1)
    def _():
        o_ref[...] = sums[...] / jnp.maximum(cnt[...], 1.0)


_pool_kernel = pl.pallas_call(
    _pool_body,
    grid=(_GRID,),
    in_specs=[pl.BlockSpec((_BLK, D), lambda i: (i, 0)),
              pl.BlockSpec((_BLK, D), lambda i: (i, 0)),
              pl.BlockSpec((_BLK, D), lambda i: (i, 0)),
              pl.BlockSpec((_BLK, D), lambda i: (i, 0)),
              pl.BlockSpec((1, D), lambda i: (0, 0)),
              pl.BlockSpec((_BLK, 128), lambda i: (i, 0))],
    out_specs=pl.BlockSpec((128, 128), lambda i: (0, 0)),
    out_shape=jax.ShapeDtypeStruct((128, 128), jnp.float32),
    scratch_shapes=[pltpu.VMEM((128, 128), jnp.float32),
                    pltpu.VMEM((128, 128), jnp.float32)],
)


def kernel(x, W1, b1, W2, b2, W3, b3, W4, b4, edge_index, batch):
    npad_e = EPAD - E
    pad_idx = (N + jnp.arange(npad_e, dtype=jnp.int32) % (NPAD - N))
    src_flat = jnp.concatenate([edge_index[0], pad_idx])
    dst_flat = jnp.concatenate([edge_index[1], pad_idx])
    combo = (src_flat | (dst_flat << 16)).reshape(NTILES, NCHUNK, CHUNK)

    x_pad = jnp.zeros((NPAD, D), jnp.float32).at[:N].set(x)
    batch_pad = jnp.full((NPAD,), G, jnp.int32).at[:N].set(batch)
    batch_b = jnp.broadcast_to(batch_pad[:, None], (NPAD, 128))

    hists = _deg_kernel(combo)
    deg = jnp.sum(hists, axis=0) + 1.0
    dinv = jnp.where(deg > 0, lax.rsqrt(jnp.maximum(deg, 1e-30)), 0.0)
    dinv = jnp.where(jnp.arange(NPAD) < N, dinv, 0.0)
    dinv_b = jnp.broadcast_to(dinv[:, None], (NPAD, D))

    t = _first_kernel(x_pad, dinv_b, W1)
    for (b, Wn) in ((b1, W2), (b2, W3), (b3, W4)):
        p0, p1 = _spmm_kernel(t, combo)
        t = _mid_kernel(p0, p1, t, dinv_b, b.reshape(1, D), Wn)
    p0, p1 = _spmm_kernel(t, combo)
    hg_full = _pool_kernel(p0, p1, t, dinv_b, b4.reshape(1, D), batch_b)
    return hg_full[:G]


# speedup vs baseline: 1.0817x; 1.0573x over previous
import functools

import jax
import jax.numpy as jnp
from jax import lax
from jax.experimental import pallas as pl
from jax.experimental.pallas import tpu as pltpu
from jax.experimental.pallas import tpu_sc as plsc

N = 10000
D = 128
G = 64
E = 320000

NPAD = 10240
NTILES = 32
CHUNK = 64
NCHUNK = 159
NBUF = 3
EPT = NCHUNK * CHUNK
EPAD = NTILES * EPT
RPS = NPAD // 16

_mesh = plsc.VectorSubcoreMesh(
    core_axis_name="c", subcore_axis_name="s", num_cores=2, num_subcores=16)


def _deg_body(combo, out, cbuf, hist):
    c = lax.axis_index("c")
    s = lax.axis_index("s")
    wid = s * 2 + c
    pltpu.sync_copy(combo.at[wid], cbuf)

    zero16 = jnp.zeros((16,), jnp.float32)

    def zrow(i, _):
        hist[pl.ds(i * 16, 16)] = zero16
        return ()
    lax.fori_loop(0, NPAD // 16, zrow, ())

    ones16 = jnp.ones((16,), jnp.float32)

    def brow(j, _):
        for k in range(CHUNK // 16):
            dv = lax.shift_right_logical(cbuf[j, pl.ds(k * 16, 16)], 16)
            plsc.addupdate_scatter(hist, [dv], ones16)
        return ()
    lax.fori_loop(0, NCHUNK, brow, ())

    pltpu.sync_copy(hist, out.at[wid])


_deg_kernel = functools.partial(
    pl.kernel,
    out_type=jax.ShapeDtypeStruct((NTILES, NPAD), jnp.float32),
    mesh=_mesh,
    scratch_types=[
        pltpu.VMEM((NCHUNK, CHUNK), jnp.int32),
        pltpu.VMEM((NPAD,), jnp.float32),
    ],
    compiler_params=pltpu.CompilerParams(needs_layout_passes=False),
)(_deg_body)


def _spmm_body(table, combo, out0, out1, acc, cbuf, sidx, didx,
               gbufs, gsems, ssems):
    c = lax.axis_index("c")
    s = lax.axis_index("s")
    wid = s * 2 + c
    pltpu.sync_copy(combo.at[wid], cbuf)

    zero16 = jnp.zeros((16,), jnp.float32)

    def zrow(i, _):
        for k in range(8):
            gbufs[0][i, pl.ds(k * 16, 16)] = zero16
        return ()
    lax.fori_loop(0, CHUNK, zrow, ())
    for k in range(RPS // CHUNK):
        pltpu.sync_copy(gbufs[0], acc.at[pl.ds(s * RPS + k * CHUNK, CHUNK)])

    def unpack(j, k):
        for q in range(CHUNK // 16):
            v = cbuf[j, pl.ds(q * 16, 16)]
            sidx[k, pl.ds(q * 16, 16)] = lax.bitwise_and(v, 0xFFFF)
            didx[k, pl.ds(q * 16, 16)] = lax.shift_right_logical(v, 16)

    for k in range(2):
        unpack(k, k)
        pltpu.async_copy(table.at[sidx.at[k]], gbufs[k], gsems[k])
    plsc.subcore_barrier()

    def body(i, _):
        for k in range(NBUF):
            jv = i * NBUF + k
            kp = (k + 2) % NBUF

            @pl.when(jv >= 1)
            def _():
                pltpu.make_async_copy(
                    gbufs[kp], acc.at[didx.at[kp]], ssems[kp]).wait()

            @pl.when(jv + 2 < NCHUNK)
            def _():
                unpack(jv + 2, kp)
                pltpu.async_copy(
                    table.at[sidx.at[kp]], gbufs[kp], gsems[kp])

            pltpu.make_async_copy(
                table.at[sidx.at[k]], gbufs[k], gsems[k]).wait()
            pltpu.async_copy(
                gbufs[k], acc.at[didx.at[k]], ssems[k], add=True)
        return ()
    lax.fori_loop(0, NCHUNK // NBUF, body, ())
    k_last = (NCHUNK - 1) % NBUF
    pltpu.make_async_copy(
        gbufs[k_last], acc.at[didx.at[k_last]], ssems[k_last]).wait()
    plsc.subcore_barrier()

    r0 = s * RPS

    @pl.when(c == 0)
    def _():
        pltpu.sync_copy(acc.at[pl.ds(r0, RPS)], out0.at[pl.ds(r0, RPS)])

    @pl.when(c == 1)
    def _():
        pltpu.sync_copy(acc.at[pl.ds(r0, RPS)], out1.at[pl.ds(r0, RPS)])


_spmm_kernel = functools.partial(
    pl.kernel,
    out_type=(jax.ShapeDtypeStruct((NPAD, D), jnp.float32),
              jax.ShapeDtypeStruct((NPAD, D), jnp.float32)),
    mesh=_mesh,
    scratch_types=[
        pltpu.VMEM_SHARED((NPAD, D), jnp.float32),
        pltpu.VMEM((NCHUNK, CHUNK), jnp.int32),
        pltpu.VMEM((NBUF, CHUNK), jnp.int32),
        pltpu.VMEM((NBUF, CHUNK), jnp.int32),
        [pltpu.VMEM((CHUNK, D), jnp.float32)] * NBUF,
        [pltpu.SemaphoreType.DMA] * NBUF,
        [pltpu.SemaphoreType.DMA] * NBUF,
    ],
    compiler_params=pltpu.CompilerParams(needs_layout_passes=False),
)(_spmm_body)


_BLK = 1280
_GRID = NPAD // _BLK


def _first_body(x_ref, dinv_ref, w_ref, o_ref):
    o_ref[...] = jnp.dot(x_ref[...], w_ref[...],
                         preferred_element_type=jnp.float32) * dinv_ref[...]


_first_kernel = pl.pallas_call(
    _first_body,
    grid=(_GRID,),
    in_specs=[pl.BlockSpec((_BLK, D), lambda i: (i, 0)),
              pl.BlockSpec((_BLK, D), lambda i: (i, 0)),
              pl.BlockSpec((D, D), lambda i: (0, 0))],
    out_specs=pl.BlockSpec((_BLK, D), lambda i: (i, 0)),
    out_shape=jax.ShapeDtypeStruct((NPAD, D), jnp.float32),
)


def _mid_body(p0_ref, p1_ref, t_ref, dinv_ref, b_ref, w_ref, o_ref):
    h = jnp.maximum(
        (p0_ref[...] + p1_ref[...] + t_ref[...]) * dinv_ref[...]
        + b_ref[...], 0.0)
    o_ref[...] = jnp.dot(h, w_ref[...],
                         preferred_element_type=jnp.float32) * dinv_ref[...]


_mid_kernel = pl.pallas_call(
    _mid_body,
    grid=(_GRID,),
    in_specs=[pl.BlockSpec((_BLK, D), lambda i: (i, 0)),
              pl.BlockSpec((_BLK, D), lambda i: (i, 0)),
              pl.BlockSpec((_BLK, D), lambda i: (i, 0)),
              pl.BlockSpec((_BLK, D), lambda i: (i, 0)),
              pl.BlockSpec((1, D), lambda i: (0, 0)),
              pl.BlockSpec((D, D), lambda i: (0, 0))],
    out_specs=pl.BlockSpec((_BLK, D), lambda i: (i, 0)),
    out_shape=jax.ShapeDtypeStruct((NPAD, D), jnp.float32),
)


def _pool_body(p0_ref, p1_ref, t_ref, dinv_ref, b_ref, bt_ref, o_ref,
               sums, cnt):
    i = pl.program_id(0)

    @pl.when(i == 0)
    def _():
        sums[...] = jnp.zeros_like(sums)
        cnt[...] = jnp.zeros_like(cnt)

    h = jnp.maximum(
        (p0_ref[...] + p1_ref[...] + t_ref[...]) * dinv_ref[...]
        + b_ref[...], 0.0)
    gid = lax.broadcasted_iota(jnp.int32, (_BLK, 128), 1)
    oh = (bt_ref[...] == gid).astype(jnp.float32)
    dn = (((0,), (0,)), ((), ()))
    sums[...] += lax.dot_general(oh, h, dn,
                                 preferred_element_type=jnp.float32)
    cnt[...] += lax.dot_general(oh, jnp.ones((_BLK, 128), jnp.float32), dn,
                                preferred_element_type=jnp.float32)

    @pl.when(i == _GRID - 1)
    def _():
        o_ref[...] = sums[...] / jnp.maximum(cnt[...], 1.0)


_pool_kernel = pl.pallas_call(
    _pool_body,
    grid=(_GRID,),
    in_specs=[pl.BlockSpec((_BLK, D), lambda i: (i, 0)),
              pl.BlockSpec((_BLK, D), lambda i: (i, 0)),
              pl.BlockSpec((_BLK, D), lambda i: (i, 0)),
              pl.BlockSpec((_BLK, D), lambda i: (i, 0)),
              pl.BlockSpec((1, D), lambda i: (0, 0)),
              pl.BlockSpec((_BLK, 128), lambda i: (i, 0))],
    out_specs=pl.BlockSpec((128, 128), lambda i: (0, 0)),
    out_shape=jax.ShapeDtypeStruct((128, 128), jnp.float32),
    scratch_shapes=[pltpu.VMEM((128, 128), jnp.float32),
                    pltpu.VMEM((128, 128), jnp.float32)],
)


def kernel(x, W1, b1, W2, b2, W3, b3, W4, b4, edge_index, batch):
    npad_e = EPAD - E
    pad_idx = (N + jnp.arange(npad_e, dtype=jnp.int32) % (NPAD - N))
    src_flat = jnp.concatenate([edge_index[0], pad_idx])
    dst_flat = jnp.concatenate([edge_index[1], pad_idx])
    combo = (src_flat | (dst_flat << 16)).reshape(NTILES, NCHUNK, CHUNK)

    x_pad = jnp.zeros((NPAD, D), jnp.float32).at[:N].set(x)
    batch_pad = jnp.full((NPAD,), G, jnp.int32).at[:N].set(batch)
    batch_b = jnp.broadcast_to(batch_pad[:, None], (NPAD, 128))

    hists = _deg_kernel(combo)
    deg = (jnp.ones((1, NTILES), jnp.float32) @ hists).reshape(NPAD) + 1.0
    dinv = jnp.where(deg > 0, lax.rsqrt(jnp.maximum(deg, 1e-30)), 0.0)
    dinv = jnp.where(jnp.arange(NPAD) < N, dinv, 0.0)
    dinv_b = jnp.broadcast_to(dinv[:, None], (NPAD, D))

    t = _first_kernel(x_pad, dinv_b, W1)
    for (b, Wn) in ((b1, W2), (b2, W3), (b3, W4)):
        p0, p1 = _spmm_kernel(t, combo)
        t = _mid_kernel(p0, p1, t, dinv_b, b.reshape(1, D), Wn)
    p0, p1 = _spmm_kernel(t, combo)
    hg_full = _pool_kernel(p0, p1, t, dinv_b, b4.reshape(1, D), batch_b)
    return hg_full[:G]


# speedup vs baseline: 1.0824x; 1.0006x over previous
import functools

import jax
import jax.numpy as jnp
from jax import lax
from jax.experimental import pallas as pl
from jax.experimental.pallas import tpu as pltpu
from jax.experimental.pallas import tpu_sc as plsc

N = 10000
D = 128
G = 64
E = 320000

NPAD = 10240
NTILES = 32
CHUNK = 64
NCHUNK = 159
NBUF = 3
EPT = NCHUNK * CHUNK
EPAD = NTILES * EPT
RPS = NPAD // 16

_mesh = plsc.VectorSubcoreMesh(
    core_axis_name="c", subcore_axis_name="s", num_cores=2, num_subcores=16)


def _deg_body(combo, out, cbuf, hist):
    c = lax.axis_index("c")
    s = lax.axis_index("s")
    wid = s * 2 + c
    pltpu.sync_copy(combo.at[wid], cbuf)

    zero16 = jnp.zeros((16,), jnp.float32)

    def zrow(i, _):
        hist[pl.ds(i * 16, 16)] = zero16
        return ()
    lax.fori_loop(0, NPAD // 16, zrow, ())

    ones16 = jnp.ones((16,), jnp.float32)

    def brow(j, _):
        for k in range(CHUNK // 16):
            dv = lax.shift_right_logical(cbuf[j, pl.ds(k * 16, 16)], 16)
            plsc.addupdate_scatter(hist, [dv], ones16)
        return ()
    lax.fori_loop(0, NCHUNK, brow, ())

    pltpu.sync_copy(hist, out.at[wid])


_deg_kernel = functools.partial(
    pl.kernel,
    out_type=jax.ShapeDtypeStruct((NTILES, NPAD), jnp.float32),
    mesh=_mesh,
    scratch_types=[
        pltpu.VMEM((NCHUNK, CHUNK), jnp.int32),
        pltpu.VMEM((NPAD,), jnp.float32),
    ],
    compiler_params=pltpu.CompilerParams(needs_layout_passes=False),
)(_deg_body)


def _spmm_body(table, combo, out0, out1, acc, cbuf, sidx, didx,
               gbufs, gsems, ssems):
    c = lax.axis_index("c")
    s = lax.axis_index("s")
    wid = s * 2 + c
    pltpu.sync_copy(combo.at[wid], cbuf)

    zero16 = jnp.zeros((16,), jnp.float32)

    def zrow(i, _):
        for k in range(8):
            gbufs[0][i, pl.ds(k * 16, 16)] = zero16
        return ()
    lax.fori_loop(0, CHUNK, zrow, ())
    for k in range(RPS // CHUNK):
        pltpu.sync_copy(gbufs[0], acc.at[pl.ds(s * RPS + k * CHUNK, CHUNK)])

    def unpack(j, k):
        for q in range(CHUNK // 16):
            v = cbuf[j, pl.ds(q * 16, 16)]
            sidx[k, pl.ds(q * 16, 16)] = lax.bitwise_and(v, 0xFFFF)
            didx[k, pl.ds(q * 16, 16)] = lax.shift_right_logical(v, 16)

    for k in range(2):
        unpack(k, k)
        pltpu.async_copy(table.at[sidx.at[k]], gbufs[k], gsems[k])
    plsc.subcore_barrier()

    def body(i, _):
        for k in range(NBUF):
            jv = i * NBUF + k
            kp = (k + 2) % NBUF

            @pl.when(jv >= 1)
            def _():
                pltpu.make_async_copy(
                    gbufs[kp], acc.at[didx.at[kp]], ssems[kp]).wait()

            @pl.when(jv + 2 < NCHUNK)
            def _():
                unpack(jv + 2, kp)
                pltpu.async_copy(
                    table.at[sidx.at[kp]], gbufs[kp], gsems[kp])

            pltpu.make_async_copy(
                table.at[sidx.at[k]], gbufs[k], gsems[k]).wait()
            pltpu.async_copy(
                gbufs[k], acc.at[didx.at[k]], ssems[k], add=True)
        return ()
    lax.fori_loop(0, NCHUNK // NBUF, body, ())
    k_last = (NCHUNK - 1) % NBUF
    pltpu.make_async_copy(
        gbufs[k_last], acc.at[didx.at[k_last]], ssems[k_last]).wait()
    plsc.subcore_barrier()

    r0 = s * RPS

    @pl.when(c == 0)
    def _():
        pltpu.sync_copy(acc.at[pl.ds(r0, RPS)], out0.at[pl.ds(r0, RPS)])

    @pl.when(c == 1)
    def _():
        pltpu.sync_copy(acc.at[pl.ds(r0, RPS)], out1.at[pl.ds(r0, RPS)])


_spmm_kernel = functools.partial(
    pl.kernel,
    out_type=(jax.ShapeDtypeStruct((NPAD, D), jnp.float32),
              jax.ShapeDtypeStruct((NPAD, D), jnp.float32)),
    mesh=_mesh,
    scratch_types=[
        pltpu.VMEM_SHARED((NPAD, D), jnp.float32),
        pltpu.VMEM((NCHUNK, CHUNK), jnp.int32),
        pltpu.VMEM((NBUF, CHUNK), jnp.int32),
        pltpu.VMEM((NBUF, CHUNK), jnp.int32),
        [pltpu.VMEM((CHUNK, D), jnp.float32)] * NBUF,
        [pltpu.SemaphoreType.DMA] * NBUF,
        [pltpu.SemaphoreType.DMA] * NBUF,
    ],
    compiler_params=pltpu.CompilerParams(needs_layout_passes=False),
)(_spmm_body)


_BLK = 1280
_GRID = NPAD // _BLK


def _first_body(x_ref, dinv_ref, w_ref, o_ref):
    o_ref[...] = jnp.dot(x_ref[...], w_ref[...],
                         preferred_element_type=jnp.float32) * dinv_ref[...]


_first_kernel = pl.pallas_call(
    _first_body,
    grid=(_GRID,),
    in_specs=[pl.BlockSpec((_BLK, D), lambda i: (i, 0)),
              pl.BlockSpec((_BLK, D), lambda i: (i, 0)),
              pl.BlockSpec((D, D), lambda i: (0, 0))],
    out_specs=pl.BlockSpec((_BLK, D), lambda i: (i, 0)),
    out_shape=jax.ShapeDtypeStruct((NPAD, D), jnp.float32),
)


def _mid_body(p0_ref, p1_ref, t_ref, dinv_ref, b_ref, w_ref, o_ref):
    h = jnp.maximum(
        (p0_ref[...] + p1_ref[...] + t_ref[...]) * dinv_ref[...]
        + b_ref[...], 0.0)
    o_ref[...] = jnp.dot(h, w_ref[...],
                         preferred_element_type=jnp.float32) * dinv_ref[...]


_mid_kernel = pl.pallas_call(
    _mid_body,
    grid=(_GRID,),
    in_specs=[pl.BlockSpec((_BLK, D), lambda i: (i, 0)),
              pl.BlockSpec((_BLK, D), lambda i: (i, 0)),
              pl.BlockSpec((_BLK, D), lambda i: (i, 0)),
              pl.BlockSpec((_BLK, D), lambda i: (i, 0)),
              pl.BlockSpec((1, D), lambda i: (0, 0)),
              pl.BlockSpec((D, D), lambda i: (0, 0))],
    out_specs=pl.BlockSpec((_BLK, D), lambda i: (i, 0)),
    out_shape=jax.ShapeDtypeStruct((NPAD, D), jnp.float32),
)


def _pool_body(p0_ref, p1_ref, t_ref, dinv_ref, b_ref, bt_ref, o_ref,
               sums, cnt):
    i = pl.program_id(0)

    @pl.when(i == 0)
    def _():
        sums[...] = jnp.zeros_like(sums)
        cnt[...] = jnp.zeros_like(cnt)

    h = jnp.maximum(
        (p0_ref[...] + p1_ref[...] + t_ref[...]) * dinv_ref[...]
        + b_ref[...], 0.0)
    gid = lax.broadcasted_iota(jnp.int32, (_BLK, 128), 1)
    oh = (bt_ref[...] == gid).astype(jnp.float32)
    dn = (((0,), (0,)), ((), ()))
    sums[...] += lax.dot_general(oh, h, dn,
                                 preferred_element_type=jnp.float32)
    cnt[...] += lax.dot_general(oh, jnp.ones((_BLK, 128), jnp.float32), dn,
                                preferred_element_type=jnp.float32)

    @pl.when(i == _GRID - 1)
    def _():
        o_ref[...] = sums[...] / jnp.maximum(cnt[...], 1.0)


_pool_kernel = pl.pallas_call(
    _pool_body,
    grid=(_GRID,),
    in_specs=[pl.BlockSpec((_BLK, D), lambda i: (i, 0)),
              pl.BlockSpec((_BLK, D), lambda i: (i, 0)),
              pl.BlockSpec((_BLK, D), lambda i: (i, 0)),
              pl.BlockSpec((_BLK, D), lambda i: (i, 0)),
              pl.BlockSpec((1, D), lambda i: (0, 0)),
              pl.BlockSpec((_BLK, D), lambda i: (i, 0))],
    out_specs=pl.BlockSpec((128, 128), lambda i: (0, 0)),
    out_shape=jax.ShapeDtypeStruct((128, 128), jnp.float32),
    scratch_shapes=[pltpu.VMEM((128, 128), jnp.float32),
                    pltpu.VMEM((128, 128), jnp.float32)],
)


def kernel(x, W1, b1, W2, b2, W3, b3, W4, b4, edge_index, batch):
    npad_e = EPAD - E
    pad_idx = (N + jnp.arange(npad_e, dtype=jnp.int32) % (NPAD - N))
    src_flat = jnp.concatenate([edge_index[0], pad_idx])
    dst_flat = jnp.concatenate([edge_index[1], pad_idx])
    combo = (src_flat | (dst_flat << 16)).reshape(NTILES, NCHUNK, CHUNK)

    x_pad = jnp.zeros((NPAD, D), jnp.float32).at[:N].set(x)
    batch_pad = jnp.full((NPAD,), G, jnp.int32).at[:N].set(batch)
    batch_b = jnp.broadcast_to(batch_pad[:, None], (NPAD, 128))

    hists = _deg_kernel(combo)
    deg = (jnp.ones((1, NTILES), jnp.float32) @ hists).reshape(NPAD) + 1.0
    dinv = jnp.where(deg > 0, lax.rsqrt(jnp.maximum(deg, 1e-30)), 0.0)
    dinv = jnp.where(jnp.arange(NPAD) < N, dinv, 0.0)
    dinv_b = jnp.broadcast_to(dinv[:, None], (NPAD, D))

    t = _first_kernel(x_pad, dinv_b, W1)
    for (b, Wn) in ((b1, W2), (b2, W3), (b3, W4)):
        p0, p1 = _spmm_kernel(t, combo)
        t = _mid_kernel(p0, p1, t, dinv_b, b.reshape(1, D), Wn)
    p0, p1 = _spmm_kernel(t, combo)
    hg_full = _pool_kernel(p0, p1, t, dinv_b, b4.reshape(1, D), batch_b)
    return hg_full[:G]
